# baseline (speedup 1.0000x reference)
import functools

import jax
import jax.numpy as jnp
from jax import lax
from jax.experimental import pallas as pl
from jax.experimental.pallas import tpu as pltpu
from jax.experimental.pallas import tpu_sc as plsc

N = 131328
B = 32
NC = 2
NS = 16
NW = NC * NS
L = 16
CH = N // NW
TILES = 33
SPAN = TILES * 128
THIRD = SPAN // 3
NTC = N // 128
NG = B // 8
NU = 3 * NG
NBUF = 4
CAPW = THIRD // L + 2
CAPV = CAPW + 3
SENT = jnp.int32(THIRD)


def _sc_body(pred_hbm, target_hbm, mask_hbm, l0_hbm, out_hbm,
             l0_v, win_v, lidx0, lidx1, lidx2, acc_v,
             p0, p1, p2, p3, t0, t1, t2, t3,
             psem0, psem1, psem2, psem3, tsem0, tsem1, tsem2, tsem3):
    m_len = mask_hbm.shape[0]
    nvm = m_len // L

    wid = lax.axis_index("s") * NC + lax.axis_index("c")
    c0 = wid * CH
    tc_s = jnp.minimum((c0 // 128), NTC - TILES)
    cs = tc_s * 128

    pbufs, tbufs = (p0, p1, p2, p3), (t0, t1, t2, t3)
    psems = (psem0, psem1, psem2, psem3)
    tsems = (tsem0, tsem1, tsem2, tsem3)
    lidxs = (lidx0, lidx1, lidx2)

    def unit_src(i):
        t, g = divmod(i, NG)
        return (pl.ds(8 * g, 8), pl.ds(cs + t * THIRD, THIRD))

    def start_unit(i):
        rs, cs_ = unit_src(i)
        u = i % NBUF
        pltpu.async_copy(pred_hbm.at[rs, cs_], pbufs[u], psems[u])
        pltpu.async_copy(target_hbm.at[rs, cs_], tbufs[u], tsems[u])

    def wait_unit(i):
        rs, cs_ = unit_src(i)
        u = i % NBUF
        pltpu.make_async_copy(pred_hbm.at[rs, cs_], pbufs[u], psems[u]).wait()
        pltpu.make_async_copy(target_hbm.at[rs, cs_], tbufs[u], tsems[u]).wait()

    for i in range(NBUF - 1):
        start_unit(i)

    pltpu.sync_copy(l0_hbm, l0_v)

    n_steps = max(1, (nvm + 1).bit_length())

    def lower_bound_vec(x):
        def step(_, carry):
            pos, rem = carry
            half = rem // 2
            mid = pos + half
            probe = jnp.min(l0_v[pl.ds(jnp.minimum(mid, nvm - 1), L)])
            active = rem > 0
            go_right = active & (probe < x)
            pos = jnp.where(go_right, mid + 1, pos)
            rem = jnp.where(active,
                            jnp.where(go_right, rem - half - 1, half),
                            0)
            return pos, rem

        pos, _ = lax.fori_loop(0, n_steps, step,
                               (jnp.int32(0), jnp.int32(nvm)))
        return pos

    bounds = [c0, cs + THIRD, cs + 2 * THIRD, c0 + CH]
    jpos = [lower_bound_vec(b) for b in bounds]
    jlos = [jnp.maximum(jpos[t] - 1, 0) for t in range(3)]
    nvs = [jpos[t + 1] - jlos[t] for t in range(3)]

    pad = jnp.full((L,), SENT, jnp.int32)
    for t in range(3):
        lo_col = bounds[t] if t == 0 else jnp.maximum(bounds[t], c0)
        hi_col = jnp.minimum(bounds[t + 1], c0 + CH)
        base = cs + t * THIRD
        lidx = lidxs[t]
        wstart = jnp.minimum(jlos[t], nvm - CAPW)
        pltpu.sync_copy(mask_hbm.at[pl.ds(wstart * L, CAPW * L)], win_v)
        rel = jlos[t] - wstart

        def build_body(k, carry, _lidx=lidx, _rel=rel, _lo=lo_col,
                       _hi=hi_col, _base=base):
            mv = win_v[pl.ds((_rel + k) * L, L)]
            valid = (mv >= _lo) & (mv < _hi)
            _lidx[pl.ds(k * L, L)] = jnp.where(valid, mv - _base, SENT)
            return carry

        lax.fori_loop(0, nvs[t], build_body, 0)
        for e in range(3):
            lidx[pl.ds((nvs[t] + e) * L, L)] = pad

    n4s = [(nvs[t] + 3) // 4 for t in range(3)]
    zero = jnp.zeros((L,), jnp.float32)

    acc = zero
    for i in range(NU):
        if i + NBUF - 1 < NU:
            start_unit(i + NBUF - 1)
        wait_unit(i)
        u = i % NBUF
        t = i // NG

        def row_body(rr, a, _lidx=lidxs[t], _n4=n4s[t], _u=u):
            row_v = jnp.full((L,), 0, jnp.int32) + rr

            def vec4_body(k, aa):
                for q in range(4):
                    cv = _lidx[pl.ds((k * 4 + q) * L, L)]
                    ok = cv < THIRD
                    cc = jnp.minimum(cv, THIRD - 1)
                    p = plsc.load_gather(pbufs[_u], [row_v, cc])
                    tt = plsc.load_gather(tbufs[_u], [row_v, cc])
                    aa = aa + jnp.where(ok, jnp.abs(p - tt), 0.0)
                return aa

            return lax.fori_loop(0, _n4, vec4_body, a)

        acc = lax.fori_loop(0, 8, row_body, acc)

    acc_v[...] = acc
    pltpu.sync_copy(acc_v, out_hbm.at[wid])


@functools.partial(jax.jit, static_argnames=("m_pad",))
def _run(pred2d, target2d, mask, l0, m_pad):
    mesh = plsc.VectorSubcoreMesh(
        core_axis_name="c", subcore_axis_name="s", num_cores=NC,
        num_subcores=NS)
    k = pl.kernel(
        _sc_body,
        out_type=jax.ShapeDtypeStruct((NW, L), jnp.float32),
        mesh=mesh,
        compiler_params=pltpu.CompilerParams(
            needs_layout_passes=False,
            disable_bounds_checks=True,
            disable_semaphore_checks=True,
            skip_device_barrier=True,
        ),
        scratch_types=[
            pltpu.VMEM((m_pad // L + L,), jnp.int32),
            pltpu.VMEM((CAPW * L,), jnp.int32),
            pltpu.VMEM((CAPV * L,), jnp.int32),
            pltpu.VMEM((CAPV * L,), jnp.int32),
            pltpu.VMEM((CAPV * L,), jnp.int32),
            pltpu.VMEM((L,), jnp.float32),
        ] + [pltpu.VMEM((8, THIRD), jnp.float32) for _ in range(2 * NBUF)]
          + [pltpu.SemaphoreType.DMA for _ in range(2 * NBUF)],
    )
    return k(pred2d, target2d, mask, l0)


def kernel(pred, target, mask):
    m = mask.shape[0]
    m_pad = ((m + L - 1) // L) * L
    m_pad = max(m_pad, CAPW * L)
    if m_pad != m:
        mask = jnp.pad(mask, (0, m_pad - m), constant_values=N)
    l0 = jnp.pad(mask[::L], (0, L), constant_values=N)
    partial = _run(pred, target, mask, l0, m_pad)
    return jnp.sum(partial)


# speedup vs baseline: 1.0023x; 1.0023x over previous
import functools

import jax
import jax.numpy as jnp
from jax import lax
from jax.experimental import pallas as pl
from jax.experimental.pallas import tpu as pltpu
from jax.experimental.pallas import tpu_sc as plsc

N = 131328
B = 32
NC = 2
NS = 16
NW = NC * NS
L = 16
CH = N // NW
TILES = 33
SPAN = TILES * 128
THIRD = SPAN // 3
NTC = N // 128
NG = B // 8
NU = 3 * NG
NBUF = 4
CAPW = THIRD // L + 2
CAPV = CAPW + 3
SENT = jnp.int32(THIRD)


def _sc_body(pred_hbm, target_hbm, mask_hbm, l0_hbm, out_hbm,
             l0_v, win_v, lidx0, lidx1, lidx2, acc_v,
             p0, p1, p2, p3, t0, t1, t2, t3,
             psem0, psem1, psem2, psem3, tsem0, tsem1, tsem2, tsem3):
    m_len = mask_hbm.shape[0]
    nvm = m_len // L

    wid = lax.axis_index("s") * NC + lax.axis_index("c")
    c0 = wid * CH
    tc_s = jnp.minimum((c0 // 128), NTC - TILES)
    cs = tc_s * 128

    pbufs, tbufs = (p0, p1, p2, p3), (t0, t1, t2, t3)
    psems = (psem0, psem1, psem2, psem3)
    tsems = (tsem0, tsem1, tsem2, tsem3)
    lidxs = (lidx0, lidx1, lidx2)

    def unit_src(i):
        t, g = divmod(i, NG)
        return (pl.ds(8 * g, 8), pl.ds(cs + t * THIRD, THIRD))

    def start_unit(i):
        rs, cs_ = unit_src(i)
        u = i % NBUF
        pltpu.async_copy(pred_hbm.at[rs, cs_], pbufs[u], psems[u])
        pltpu.async_copy(target_hbm.at[rs, cs_], tbufs[u], tsems[u])

    def wait_unit(i):
        rs, cs_ = unit_src(i)
        u = i % NBUF
        pltpu.make_async_copy(pred_hbm.at[rs, cs_], pbufs[u], psems[u]).wait()
        pltpu.make_async_copy(target_hbm.at[rs, cs_], tbufs[u], tsems[u]).wait()

    for i in range(NBUF - 1):
        start_unit(i)

    pltpu.sync_copy(l0_hbm, l0_v)

    n_steps = max(1, (nvm + 1).bit_length())

    def lower_bound_vec(x):
        def step(_, carry):
            pos, rem = carry
            half = rem // 2
            mid = pos + half
            probe = jnp.min(l0_v[pl.ds(jnp.minimum(mid, nvm - 1), L)])
            active = rem > 0
            go_right = active & (probe < x)
            pos = jnp.where(go_right, mid + 1, pos)
            rem = jnp.where(active,
                            jnp.where(go_right, rem - half - 1, half),
                            0)
            return pos, rem

        pos, _ = lax.fori_loop(0, n_steps, step,
                               (jnp.int32(0), jnp.int32(nvm)))
        return pos

    bounds = [c0, cs + THIRD, cs + 2 * THIRD, c0 + CH]
    jpos = [lower_bound_vec(b) for b in bounds]
    jlos = [jnp.maximum(jpos[t] - 1, 0) for t in range(3)]
    nvs = [jpos[t + 1] - jlos[t] for t in range(3)]

    pad = jnp.full((L,), SENT, jnp.int32)
    for t in range(3):
        lo_col = bounds[t] if t == 0 else jnp.maximum(bounds[t], c0)
        hi_col = jnp.minimum(bounds[t + 1], c0 + CH)
        base = cs + t * THIRD
        lidx = lidxs[t]
        wstart = jnp.minimum(jlos[t], nvm - CAPW)
        pltpu.sync_copy(mask_hbm.at[pl.ds(wstart * L, CAPW * L)], win_v)
        rel = jlos[t] - wstart

        def build_body(k, carry, _lidx=lidx, _rel=rel, _lo=lo_col,
                       _hi=hi_col, _base=base):
            mv = win_v[pl.ds((_rel + k) * L, L)]
            valid = (mv >= _lo) & (mv < _hi)
            _lidx[pl.ds(k * L, L)] = jnp.where(valid, mv - _base, SENT)
            return carry

        lax.fori_loop(0, nvs[t], build_body, 0)
        for e in range(3):
            lidx[pl.ds((nvs[t] + e) * L, L)] = pad

    n4s = [(nvs[t] + 3) // 4 for t in range(3)]
    zero = jnp.zeros((L,), jnp.float32)

    acc = zero
    for i in range(NU):
        if i + NBUF - 1 < NU:
            start_unit(i + NBUF - 1)
        wait_unit(i)
        u = i % NBUF
        t = i // NG

        def row_body(rr, a, _lidx=lidxs[t], _n4=n4s[t], _u=u):
            row_v = jnp.full((L,), 0, jnp.int32) + rr

            def vec4_body(k, aa):
                for q in range(4):
                    cv = _lidx[pl.ds((k * 4 + q) * L, L)]
                    ok = cv < THIRD
                    cc = jnp.minimum(cv, THIRD - 1)
                    p = plsc.load_gather(pbufs[_u], [row_v, cc])
                    tt = plsc.load_gather(tbufs[_u], [row_v, cc])
                    aa = aa + jnp.where(ok, jnp.abs(p - tt), 0.0)
                return aa

            return lax.fori_loop(0, _n4, vec4_body, a)

        acc = lax.fori_loop(0, 8, row_body, acc)

    acc_v[...] = acc
    pltpu.sync_copy(acc_v, out_hbm.at[wid])


@functools.partial(jax.jit, static_argnames=("m_pad",))
def _run(pred2d, target2d, mask, l0, m_pad):
    mesh = plsc.VectorSubcoreMesh(
        core_axis_name="c", subcore_axis_name="s", num_cores=NC,
        num_subcores=NS)
    k = pl.kernel(
        _sc_body,
        out_type=jax.ShapeDtypeStruct((NW, L), jnp.float32),
        mesh=mesh,
        compiler_params=pltpu.CompilerParams(needs_layout_passes=False),
        scratch_types=[
            pltpu.VMEM((m_pad // L + L,), jnp.int32),
            pltpu.VMEM((CAPW * L,), jnp.int32),
            pltpu.VMEM((CAPV * L,), jnp.int32),
            pltpu.VMEM((CAPV * L,), jnp.int32),
            pltpu.VMEM((CAPV * L,), jnp.int32),
            pltpu.VMEM((L,), jnp.float32),
        ] + [pltpu.VMEM((8, THIRD), jnp.float32) for _ in range(2 * NBUF)]
          + [pltpu.SemaphoreType.DMA for _ in range(2 * NBUF)],
    )
    return k(pred2d, target2d, mask, l0)


def kernel(pred, target, mask):
    m = mask.shape[0]
    m_pad = ((m + L - 1) // L) * L
    m_pad = max(m_pad, CAPW * L)
    if m_pad != m:
        mask = jnp.pad(mask, (0, m_pad - m), constant_values=N)
    l0 = jnp.pad(mask[::L], (0, L), constant_values=N)
    partial = _run(pred, target, mask, l0, m_pad)
    return jnp.sum(partial)
